# initial kernel scaffold (unmeasured)
import jax
import jax.numpy as jnp
from jax import lax
from jax.experimental import pallas as pl
from jax.experimental.pallas import tpu as pltpu

N_DEV = 16


def kernel(x, w_mat, scale_x, scale_w):
    m, k_shard = x.shape
    _, n_out = w_mat.shape
    chunk = m // N_DEV

    def body(x_ref, w_ref, sx_ref, sw_ref, out_ref,
             comm_ref, send_sems, recv_sems, wb_ref, credit_sem):
        my = lax.axis_index("i")
        left = (my - 1) % N_DEV
        right = (my + 1) % N_DEV

        barrier = pltpu.get_barrier_semaphore()
        for nbr in (left, right):
            pl.semaphore_signal(barrier, inc=1, device_id=(nbr,),
                                device_id_type=pl.DeviceIdType.MESH)
        pl.semaphore_wait(barrier, 2)

        wb_ref[...] = w_ref[...].astype(jnp.bfloat16)
        s = sx_ref[0] * sw_ref[0]

        def partial(c):
            xa = x_ref[pl.ds(c * chunk, chunk), :].astype(jnp.bfloat16)
            return lax.dot_general(
                xa, wb_ref[...],
                dimension_numbers=(((1,), (0,)), ((), ())),
                preferred_element_type=jnp.float32)

        out_ref[pl.ds(my * chunk, chunk), :] = partial(my)

        for h in range(2 * (N_DEV - 1)):
            slot = h % 2
            rs = h < N_DEV - 1
            if rs:
                c_send = (my - h) % N_DEV
                c_recv = (my - 1 - h) % N_DEV
            else:
                g = h - (N_DEV - 1)
                c_send = (my + 1 - g) % N_DEV
                c_recv = (my - g) % N_DEV

            if h >= 2:
                pl.semaphore_wait(credit_sem, 1)
            rdma = pltpu.make_async_remote_copy(
                src_ref=out_ref.at[pl.ds(c_send * chunk, chunk), :],
                dst_ref=comm_ref.at[slot],
                send_sem=send_sems.at[slot],
                recv_sem=recv_sems.at[slot],
                device_id=(right,),
                device_id_type=pl.DeviceIdType.MESH,
            )
            rdma.start()
            if rs:
                p = partial(c_recv)
            rdma.wait()
            if rs:
                val = comm_ref[slot] + p
                if h == N_DEV - 2:
                    y = val * s
                    val = y * (1.0 / (1.0 + jnp.exp(-y)))
                out_ref[pl.ds(c_recv * chunk, chunk), :] = val
            else:
                out_ref[pl.ds(c_recv * chunk, chunk), :] = comm_ref[slot]
            pl.semaphore_signal(credit_sem, inc=1, device_id=(left,),
                                device_id_type=pl.DeviceIdType.MESH)

        pl.semaphore_wait(credit_sem, 2)

    return pl.pallas_call(
        body,
        out_shape=jax.ShapeDtypeStruct((m, n_out), jnp.float32),
        in_specs=[
            pl.BlockSpec(memory_space=pltpu.VMEM),
            pl.BlockSpec(memory_space=pltpu.VMEM),
            pl.BlockSpec(memory_space=pltpu.SMEM),
            pl.BlockSpec(memory_space=pltpu.SMEM),
        ],
        out_specs=pl.BlockSpec(memory_space=pltpu.VMEM),
        scratch_shapes=[
            pltpu.VMEM((2, chunk, n_out), jnp.float32),
            pltpu.SemaphoreType.DMA((2,)),
            pltpu.SemaphoreType.DMA((2,)),
            pltpu.VMEM((k_shard, n_out), jnp.bfloat16),
            pltpu.SemaphoreType.REGULAR,
        ],
        compiler_params=pltpu.CompilerParams(collective_id=0),
    )(x, w_mat, scale_x, scale_w)


# baseline (device time: 777477 ns/iter reference)
import jax
import jax.numpy as jnp
from jax import lax
from jax.experimental import pallas as pl
from jax.experimental.pallas import tpu as pltpu

N_DEV = 16


def kernel(x, w_mat, scale_x, scale_w):
    m, k_shard = x.shape
    _, n_out = w_mat.shape
    chunk = m // N_DEV

    def body(x_ref, w_ref, sx_ref, sw_ref, out_ref,
             comm_ref, send_sems, recv_sems, wb_ref, credit_sem):
        my = lax.axis_index("i")
        left = (my - 1) % N_DEV
        right = (my + 1) % N_DEV

        barrier = pltpu.get_barrier_semaphore()
        for nbr in (left, right):
            pl.semaphore_signal(barrier, inc=1, device_id=(nbr,),
                                device_id_type=pl.DeviceIdType.MESH)
        pl.semaphore_wait(barrier, 2)

        wb_ref[...] = w_ref[...].astype(jnp.bfloat16)
        s = sx_ref[0] * sw_ref[0]

        def partial(c):
            xa = x_ref[pl.ds(c * chunk, chunk), :].astype(jnp.bfloat16)
            return lax.dot_general(
                xa, wb_ref[...],
                dimension_numbers=(((1,), (0,)), ((), ())),
                preferred_element_type=jnp.float32)

        out_ref[pl.ds(my * chunk, chunk), :] = partial(my)

        for h in range(2 * (N_DEV - 1)):
            slot = h % 2
            rs = h < N_DEV - 1
            if rs:
                c_send = (my - h) % N_DEV
                c_recv = (my - 1 - h) % N_DEV
            else:
                g = h - (N_DEV - 1)
                c_send = (my + 1 - g) % N_DEV
                c_recv = (my - g) % N_DEV

            if h >= 2:
                pl.semaphore_wait(credit_sem, 1)
            rdma = pltpu.make_async_remote_copy(
                src_ref=out_ref.at[pl.ds(c_send * chunk, chunk), :],
                dst_ref=comm_ref.at[slot],
                send_sem=send_sems.at[slot],
                recv_sem=recv_sems.at[slot],
                device_id=(right,),
                device_id_type=pl.DeviceIdType.MESH,
            )
            rdma.start()
            if rs:
                p = partial(c_recv)
            rdma.wait()
            if rs:
                val = comm_ref[slot] + p
                if h == N_DEV - 2:
                    y = val * s
                    val = y * (1.0 / (1.0 + jnp.exp(-y)))
                out_ref[pl.ds(c_recv * chunk, chunk), :] = val
            else:
                out_ref[pl.ds(c_recv * chunk, chunk), :] = comm_ref[slot]
            pl.semaphore_signal(credit_sem, inc=1, device_id=(left,),
                                device_id_type=pl.DeviceIdType.MESH)

        pl.semaphore_wait(credit_sem, 2)

    return pl.pallas_call(
        body,
        out_shape=jax.ShapeDtypeStruct((m, n_out), jnp.float32),
        in_specs=[
            pl.BlockSpec(memory_space=pltpu.VMEM),
            pl.BlockSpec(memory_space=pltpu.VMEM),
            pl.BlockSpec(memory_space=pltpu.SMEM),
            pl.BlockSpec(memory_space=pltpu.SMEM),
        ],
        out_specs=pl.BlockSpec(memory_space=pltpu.VMEM),
        scratch_shapes=[
            pltpu.VMEM((2, chunk, n_out), jnp.float32),
            pltpu.SemaphoreType.DMA((2,)),
            pltpu.SemaphoreType.DMA((2,)),
            pltpu.VMEM((k_shard, n_out), jnp.bfloat16),
            pltpu.SemaphoreType.REGULAR,
        ],
        compiler_params=pltpu.CompilerParams(
            collective_id=0,
            vmem_limit_bytes=100 * 1024 * 1024,
        ),
    )(x, w_mat, scale_x, scale_w)


# device time: 481293 ns/iter; 1.6154x vs baseline; 1.6154x over previous
import jax
import jax.numpy as jnp
from jax import lax
from jax.experimental import pallas as pl
from jax.experimental.pallas import tpu as pltpu

N_DEV = 16
COMM_DTYPE = jnp.float32


def kernel(x, w_mat, scale_x, scale_w):
    m, k_shard = x.shape
    _, n_out = w_mat.shape
    chunk = m // N_DEV
    half = n_out // 2

    def body(x_ref, w_ref, sx_ref, sw_ref, out_ref,
             comm_r, comm_l, stage_r, stage_l,
             send_sems_r, recv_sems_r, send_sems_l, recv_sems_l,
             wb_ref, credit_r, credit_l):
        my = lax.axis_index("i")
        left = (my - 1) % N_DEV
        right = (my + 1) % N_DEV

        barrier = pltpu.get_barrier_semaphore()
        for nbr in (left, right):
            pl.semaphore_signal(barrier, inc=1, device_id=(nbr,),
                                device_id_type=pl.DeviceIdType.MESH)
        pl.semaphore_wait(barrier, 2)

        wb_ref[...] = w_ref[...].astype(jnp.bfloat16)
        s = sx_ref[0] * sw_ref[0]

        def partial(c, lo):
            xa = x_ref[pl.ds(c * chunk, chunk), :].astype(jnp.bfloat16)
            return lax.dot_general(
                xa, wb_ref[:, lo:lo + half],
                dimension_numbers=(((1,), (0,)), ((), ())),
                preferred_element_type=jnp.float32)

        def silu(v):
            y = v * s
            return y * (1.0 / (1.0 + jnp.exp(-y)))

        stage_r[0] = partial(my, 0).astype(COMM_DTYPE)
        stage_l[0] = partial(my, half).astype(COMM_DTYPE)

        for h in range(2 * (N_DEV - 1)):
            slot = h % 2
            nslot = (h + 1) % 2
            rs = h < N_DEV - 1
            if rs:
                c_recv_r = (my - 1 - h) % N_DEV
                c_recv_l = (my + 1 + h) % N_DEV
            else:
                g = h - (N_DEV - 1)
                c_recv_r = (my - g) % N_DEV
                c_recv_l = (my + g) % N_DEV

            if h >= 2:
                pl.semaphore_wait(credit_r, 1)
                pl.semaphore_wait(credit_l, 1)
            rdma_r = pltpu.make_async_remote_copy(
                src_ref=stage_r.at[slot],
                dst_ref=comm_r.at[slot],
                send_sem=send_sems_r.at[slot],
                recv_sem=recv_sems_r.at[slot],
                device_id=(right,),
                device_id_type=pl.DeviceIdType.MESH,
            )
            rdma_l = pltpu.make_async_remote_copy(
                src_ref=stage_l.at[slot],
                dst_ref=comm_l.at[slot],
                send_sem=send_sems_l.at[slot],
                recv_sem=recv_sems_l.at[slot],
                device_id=(left,),
                device_id_type=pl.DeviceIdType.MESH,
            )
            rdma_r.start()
            rdma_l.start()
            if rs:
                p_r = partial(c_recv_r, 0)
                p_l = partial(c_recv_l, half)
            rdma_r.wait()
            rdma_l.wait()
            if rs:
                val_r = comm_r[slot].astype(jnp.float32) + p_r
                val_l = comm_l[slot].astype(jnp.float32) + p_l
                if h == N_DEV - 2:
                    val_r = silu(val_r)
                    val_l = silu(val_l)
                    out_ref[pl.ds(c_recv_r * chunk, chunk), :half] = val_r
                    out_ref[pl.ds(c_recv_l * chunk, chunk), half:] = val_l
                stage_r[nslot] = val_r.astype(COMM_DTYPE)
                stage_l[nslot] = val_l.astype(COMM_DTYPE)
            else:
                out_ref[pl.ds(c_recv_r * chunk, chunk), :half] = (
                    comm_r[slot].astype(jnp.float32))
                out_ref[pl.ds(c_recv_l * chunk, chunk), half:] = (
                    comm_l[slot].astype(jnp.float32))
                if h < 2 * (N_DEV - 1) - 1:
                    stage_r[nslot] = comm_r[slot]
                    stage_l[nslot] = comm_l[slot]
            pl.semaphore_signal(credit_r, inc=1, device_id=(left,),
                                device_id_type=pl.DeviceIdType.MESH)
            pl.semaphore_signal(credit_l, inc=1, device_id=(right,),
                                device_id_type=pl.DeviceIdType.MESH)

        pl.semaphore_wait(credit_r, 2)
        pl.semaphore_wait(credit_l, 2)

    return pl.pallas_call(
        body,
        out_shape=jax.ShapeDtypeStruct((m, n_out), jnp.float32),
        in_specs=[
            pl.BlockSpec(memory_space=pltpu.VMEM),
            pl.BlockSpec(memory_space=pltpu.VMEM),
            pl.BlockSpec(memory_space=pltpu.SMEM),
            pl.BlockSpec(memory_space=pltpu.SMEM),
        ],
        out_specs=pl.BlockSpec(memory_space=pltpu.VMEM),
        scratch_shapes=[
            pltpu.VMEM((2, chunk, half), COMM_DTYPE),
            pltpu.VMEM((2, chunk, half), COMM_DTYPE),
            pltpu.VMEM((2, chunk, half), COMM_DTYPE),
            pltpu.VMEM((2, chunk, half), COMM_DTYPE),
            pltpu.SemaphoreType.DMA((2,)),
            pltpu.SemaphoreType.DMA((2,)),
            pltpu.SemaphoreType.DMA((2,)),
            pltpu.SemaphoreType.DMA((2,)),
            pltpu.VMEM((k_shard, n_out), jnp.bfloat16),
            pltpu.SemaphoreType.REGULAR,
            pltpu.SemaphoreType.REGULAR,
        ],
        compiler_params=pltpu.CompilerParams(
            collective_id=0,
            vmem_limit_bytes=100 * 1024 * 1024,
        ),
    )(x, w_mat, scale_x, scale_w)


# device time: 311443 ns/iter; 2.4964x vs baseline; 1.5454x over previous
import jax
import jax.numpy as jnp
from jax import lax
from jax.experimental import pallas as pl
from jax.experimental.pallas import tpu as pltpu

N_DEV = 16
COMM_DTYPE = jnp.bfloat16


def kernel(x, w_mat, scale_x, scale_w):
    m, k_shard = x.shape
    _, n_out = w_mat.shape
    chunk = m // N_DEV
    half = n_out // 2

    def body(x_ref, w_ref, sx_ref, sw_ref, out_ref,
             comm_r, comm_l, stage_r, stage_l,
             send_sems_r, recv_sems_r, send_sems_l, recv_sems_l,
             wb_ref, credit_r, credit_l):
        my = lax.axis_index("i")
        left = (my - 1) % N_DEV
        right = (my + 1) % N_DEV

        barrier = pltpu.get_barrier_semaphore()
        for nbr in (left, right):
            pl.semaphore_signal(barrier, inc=1, device_id=(nbr,),
                                device_id_type=pl.DeviceIdType.MESH)
        pl.semaphore_wait(barrier, 2)

        wb_ref[...] = w_ref[...].astype(jnp.bfloat16)
        s = sx_ref[0] * sw_ref[0]

        def partial(c, lo):
            xa = x_ref[pl.ds(c * chunk, chunk), :].astype(jnp.bfloat16)
            return lax.dot_general(
                xa, wb_ref[:, lo:lo + half],
                dimension_numbers=(((1,), (0,)), ((), ())),
                preferred_element_type=jnp.float32)

        def silu(v):
            y = v * s
            return y * (1.0 / (1.0 + jnp.exp(-y)))

        stage_r[0] = partial(my, 0).astype(COMM_DTYPE)
        stage_l[0] = partial(my, half).astype(COMM_DTYPE)

        for h in range(2 * (N_DEV - 1)):
            slot = h % 2
            nslot = (h + 1) % 2
            rs = h < N_DEV - 1
            if rs:
                c_recv_r = (my - 1 - h) % N_DEV
                c_recv_l = (my + 1 + h) % N_DEV
            else:
                g = h - (N_DEV - 1)
                c_recv_r = (my - g) % N_DEV
                c_recv_l = (my + g) % N_DEV

            if h >= 2:
                pl.semaphore_wait(credit_r, 1)
                pl.semaphore_wait(credit_l, 1)
            rdma_r = pltpu.make_async_remote_copy(
                src_ref=stage_r.at[slot],
                dst_ref=comm_r.at[slot],
                send_sem=send_sems_r.at[slot],
                recv_sem=recv_sems_r.at[slot],
                device_id=(right,),
                device_id_type=pl.DeviceIdType.MESH,
            )
            rdma_l = pltpu.make_async_remote_copy(
                src_ref=stage_l.at[slot],
                dst_ref=comm_l.at[slot],
                send_sem=send_sems_l.at[slot],
                recv_sem=recv_sems_l.at[slot],
                device_id=(left,),
                device_id_type=pl.DeviceIdType.MESH,
            )
            rdma_r.start()
            rdma_l.start()
            if rs:
                p_r = partial(c_recv_r, 0)
                p_l = partial(c_recv_l, half)
            rdma_r.wait()
            rdma_l.wait()
            if rs:
                val_r = comm_r[slot].astype(jnp.float32) + p_r
                val_l = comm_l[slot].astype(jnp.float32) + p_l
                if h == N_DEV - 2:
                    val_r = silu(val_r)
                    val_l = silu(val_l)
                    out_ref[pl.ds(c_recv_r * chunk, chunk), :half] = val_r
                    out_ref[pl.ds(c_recv_l * chunk, chunk), half:] = val_l
                stage_r[nslot] = val_r.astype(COMM_DTYPE)
                stage_l[nslot] = val_l.astype(COMM_DTYPE)
            else:
                out_ref[pl.ds(c_recv_r * chunk, chunk), :half] = (
                    comm_r[slot].astype(jnp.float32))
                out_ref[pl.ds(c_recv_l * chunk, chunk), half:] = (
                    comm_l[slot].astype(jnp.float32))
                if h < 2 * (N_DEV - 1) - 1:
                    stage_r[nslot] = comm_r[slot]
                    stage_l[nslot] = comm_l[slot]
            pl.semaphore_signal(credit_r, inc=1, device_id=(left,),
                                device_id_type=pl.DeviceIdType.MESH)
            pl.semaphore_signal(credit_l, inc=1, device_id=(right,),
                                device_id_type=pl.DeviceIdType.MESH)

        pl.semaphore_wait(credit_r, 2)
        pl.semaphore_wait(credit_l, 2)

    return pl.pallas_call(
        body,
        out_shape=jax.ShapeDtypeStruct((m, n_out), jnp.float32),
        in_specs=[
            pl.BlockSpec(memory_space=pltpu.VMEM),
            pl.BlockSpec(memory_space=pltpu.VMEM),
            pl.BlockSpec(memory_space=pltpu.SMEM),
            pl.BlockSpec(memory_space=pltpu.SMEM),
        ],
        out_specs=pl.BlockSpec(memory_space=pltpu.VMEM),
        scratch_shapes=[
            pltpu.VMEM((2, chunk, half), COMM_DTYPE),
            pltpu.VMEM((2, chunk, half), COMM_DTYPE),
            pltpu.VMEM((2, chunk, half), COMM_DTYPE),
            pltpu.VMEM((2, chunk, half), COMM_DTYPE),
            pltpu.SemaphoreType.DMA((2,)),
            pltpu.SemaphoreType.DMA((2,)),
            pltpu.SemaphoreType.DMA((2,)),
            pltpu.SemaphoreType.DMA((2,)),
            pltpu.VMEM((k_shard, n_out), jnp.bfloat16),
            pltpu.SemaphoreType.REGULAR,
            pltpu.SemaphoreType.REGULAR,
        ],
        compiler_params=pltpu.CompilerParams(
            collective_id=0,
            vmem_limit_bytes=100 * 1024 * 1024,
        ),
    )(x, w_mat, scale_x, scale_w)


# device time: 230861 ns/iter; 3.3677x vs baseline; 1.3490x over previous
import jax
import jax.numpy as jnp
from jax import lax
from jax.experimental import pallas as pl
from jax.experimental.pallas import tpu as pltpu

N_DEV = 16
COMM_DTYPE = jnp.bfloat16
N_SUB = 2


def kernel(x, w_mat, scale_x, scale_w):
    m, k_shard = x.shape
    _, n_out = w_mat.shape
    chunk = m // N_DEV
    n_lanes = 2 * N_SUB
    sub = n_out // n_lanes
    n_hops = 2 * (N_DEV - 1)

    def body(x_ref, w_ref, sx_ref, sw_ref, out_ref,
             comm_ref, stage_ref, send_sems, recv_sems, wb_ref, credits):
        my = lax.axis_index("i")
        left = (my - 1) % N_DEV
        right = (my + 1) % N_DEV

        lanes = []
        for li in range(n_lanes):
            d = 1 if li < N_SUB else -1
            to_peer = right if d == 1 else left
            up_peer = left if d == 1 else right
            lanes.append((li * sub, to_peer, up_peer, d))

        barrier = pltpu.get_barrier_semaphore()
        for nbr in (left, right):
            pl.semaphore_signal(barrier, inc=1, device_id=(nbr,),
                                device_id_type=pl.DeviceIdType.MESH)
        pl.semaphore_wait(barrier, 2)

        wb_ref[...] = w_ref[...].astype(jnp.bfloat16)
        s = sx_ref[0] * sw_ref[0]

        def partial(c, lo):
            xa = x_ref[pl.ds(c * chunk, chunk), :].astype(jnp.bfloat16)
            return lax.dot_general(
                xa, wb_ref[:, lo:lo + sub],
                dimension_numbers=(((1,), (0,)), ((), ())),
                preferred_element_type=jnp.float32)

        def silu(v):
            y = v * s
            return y * (1.0 / (1.0 + jnp.exp(-y)))

        def desc(li, slot, peer):
            return pltpu.make_async_remote_copy(
                src_ref=stage_ref.at[li, slot],
                dst_ref=comm_ref.at[li, slot],
                send_sem=send_sems.at[li, slot],
                recv_sem=recv_sems.at[li, slot],
                device_id=(peer,),
                device_id_type=pl.DeviceIdType.MESH,
            )

        for li, (lo, to_peer, _up, _d) in enumerate(lanes):
            stage_ref[li, 0] = partial(my, lo).astype(COMM_DTYPE)
            desc(li, 0, to_peer).start()

        for h in range(n_hops):
            slot = h % 2
            nslot = (h + 1) % 2
            rs = h < N_DEV - 1
            g = h - (N_DEV - 1)
            for li, (lo, to_peer, up_peer, d) in enumerate(lanes):
                c_recv = (my - d * (1 + h if rs else g)) % N_DEV

                if h >= 1:
                    desc(li, nslot, to_peer).wait_send()
                desc(li, slot, to_peer).wait_recv()

                if rs:
                    val = (comm_ref[li, slot].astype(jnp.float32)
                           + partial(c_recv, lo))
                    if h == N_DEV - 2:
                        val = silu(val)
                        out_ref[pl.ds(c_recv * chunk, chunk),
                                lo:lo + sub] = val
                    stage_ref[li, nslot] = val.astype(COMM_DTYPE)
                else:
                    out_ref[pl.ds(c_recv * chunk, chunk), lo:lo + sub] = (
                        comm_ref[li, slot].astype(jnp.float32))
                    if h < n_hops - 1:
                        stage_ref[li, nslot] = comm_ref[li, slot]

                if h < n_hops - 1:
                    if h + 1 >= 2:
                        pl.semaphore_wait(credits.at[li], 1)
                    desc(li, nslot, to_peer).start()
                pl.semaphore_signal(credits.at[li], inc=1,
                                    device_id=(up_peer,),
                                    device_id_type=pl.DeviceIdType.MESH)

        for li, (_lo, to_peer, _up, _d) in enumerate(lanes):
            desc(li, (n_hops - 1) % 2, to_peer).wait_send()
            pl.semaphore_wait(credits.at[li], 2)

    return pl.pallas_call(
        body,
        out_shape=jax.ShapeDtypeStruct((m, n_out), jnp.float32),
        in_specs=[
            pl.BlockSpec(memory_space=pltpu.VMEM),
            pl.BlockSpec(memory_space=pltpu.VMEM),
            pl.BlockSpec(memory_space=pltpu.SMEM),
            pl.BlockSpec(memory_space=pltpu.SMEM),
        ],
        out_specs=pl.BlockSpec(memory_space=pltpu.VMEM),
        scratch_shapes=[
            pltpu.VMEM((n_lanes, 2, chunk, sub), COMM_DTYPE),
            pltpu.VMEM((n_lanes, 2, chunk, sub), COMM_DTYPE),
            pltpu.SemaphoreType.DMA((n_lanes, 2)),
            pltpu.SemaphoreType.DMA((n_lanes, 2)),
            pltpu.VMEM((k_shard, n_out), jnp.bfloat16),
            pltpu.SemaphoreType.REGULAR((n_lanes,)),
        ],
        compiler_params=pltpu.CompilerParams(
            collective_id=0,
            vmem_limit_bytes=100 * 1024 * 1024,
        ),
    )(x, w_mat, scale_x, scale_w)


# device time: 229555 ns/iter; 3.3869x vs baseline; 1.0057x over previous
import jax
import jax.numpy as jnp
from jax import lax
from jax.experimental import pallas as pl
from jax.experimental.pallas import tpu as pltpu

N_DEV = 16
COMM_DTYPE = jnp.bfloat16
N_SUB = 4


def kernel(x, w_mat, scale_x, scale_w):
    m, k_shard = x.shape
    _, n_out = w_mat.shape
    chunk = m // N_DEV
    n_lanes = 2 * N_SUB
    sub = n_out // n_lanes
    n_hops = 2 * (N_DEV - 1)

    def body(x_ref, w_ref, sx_ref, sw_ref, out_ref,
             comm_ref, stage_ref, send_sems, recv_sems, wb_ref, credits):
        my = lax.axis_index("i")
        left = (my - 1) % N_DEV
        right = (my + 1) % N_DEV

        lanes = []
        for li in range(n_lanes):
            d = 1 if li < N_SUB else -1
            to_peer = right if d == 1 else left
            up_peer = left if d == 1 else right
            lanes.append((li * sub, to_peer, up_peer, d))

        barrier = pltpu.get_barrier_semaphore()
        for nbr in (left, right):
            pl.semaphore_signal(barrier, inc=1, device_id=(nbr,),
                                device_id_type=pl.DeviceIdType.MESH)
        pl.semaphore_wait(barrier, 2)

        wb_ref[...] = w_ref[...].astype(jnp.bfloat16)
        s = sx_ref[0] * sw_ref[0]

        def partial(c, lo):
            xa = x_ref[pl.ds(c * chunk, chunk), :].astype(jnp.bfloat16)
            return lax.dot_general(
                xa, wb_ref[:, lo:lo + sub],
                dimension_numbers=(((1,), (0,)), ((), ())),
                preferred_element_type=jnp.float32)

        def silu(v):
            y = v * s
            return y * (1.0 / (1.0 + jnp.exp(-y)))

        def desc(li, slot, peer):
            return pltpu.make_async_remote_copy(
                src_ref=stage_ref.at[li, slot],
                dst_ref=comm_ref.at[li, slot],
                send_sem=send_sems.at[li, slot],
                recv_sem=recv_sems.at[li, slot],
                device_id=(peer,),
                device_id_type=pl.DeviceIdType.MESH,
            )

        for li, (lo, to_peer, _up, _d) in enumerate(lanes):
            stage_ref[li, 0] = partial(my, lo).astype(COMM_DTYPE)
            desc(li, 0, to_peer).start()

        for h in range(n_hops):
            slot = h % 2
            nslot = (h + 1) % 2
            rs = h < N_DEV - 1
            g = h - (N_DEV - 1)
            for li, (lo, to_peer, up_peer, d) in enumerate(lanes):
                c_recv = (my - d * (1 + h if rs else g)) % N_DEV

                if h >= 1:
                    desc(li, nslot, to_peer).wait_send()
                desc(li, slot, to_peer).wait_recv()

                if rs:
                    val = (comm_ref[li, slot].astype(jnp.float32)
                           + partial(c_recv, lo))
                    if h == N_DEV - 2:
                        val = silu(val)
                        out_ref[pl.ds(c_recv * chunk, chunk),
                                lo:lo + sub] = val
                    stage_ref[li, nslot] = val.astype(COMM_DTYPE)
                else:
                    out_ref[pl.ds(c_recv * chunk, chunk), lo:lo + sub] = (
                        comm_ref[li, slot].astype(jnp.float32))
                    if h < n_hops - 1:
                        stage_ref[li, nslot] = comm_ref[li, slot]

                if h < n_hops - 1:
                    if h + 1 >= 2:
                        pl.semaphore_wait(credits.at[li], 1)
                    desc(li, nslot, to_peer).start()
                pl.semaphore_signal(credits.at[li], inc=1,
                                    device_id=(up_peer,),
                                    device_id_type=pl.DeviceIdType.MESH)

        for li, (_lo, to_peer, _up, _d) in enumerate(lanes):
            desc(li, (n_hops - 1) % 2, to_peer).wait_send()
            pl.semaphore_wait(credits.at[li], 2)

    return pl.pallas_call(
        body,
        out_shape=jax.ShapeDtypeStruct((m, n_out), jnp.float32),
        in_specs=[
            pl.BlockSpec(memory_space=pltpu.VMEM),
            pl.BlockSpec(memory_space=pltpu.VMEM),
            pl.BlockSpec(memory_space=pltpu.SMEM),
            pl.BlockSpec(memory_space=pltpu.SMEM),
        ],
        out_specs=pl.BlockSpec(memory_space=pltpu.VMEM),
        scratch_shapes=[
            pltpu.VMEM((n_lanes, 2, chunk, sub), COMM_DTYPE),
            pltpu.VMEM((n_lanes, 2, chunk, sub), COMM_DTYPE),
            pltpu.SemaphoreType.DMA((n_lanes, 2)),
            pltpu.SemaphoreType.DMA((n_lanes, 2)),
            pltpu.VMEM((k_shard, n_out), jnp.bfloat16),
            pltpu.SemaphoreType.REGULAR((n_lanes,)),
        ],
        compiler_params=pltpu.CompilerParams(
            collective_id=0,
            vmem_limit_bytes=100 * 1024 * 1024,
        ),
    )(x, w_mat, scale_x, scale_w)
